# initial kernel scaffold (unmeasured)
import jax
import jax.numpy as jnp
from jax import lax
from jax.experimental import pallas as pl
from jax.experimental.pallas import tpu as pltpu


def kernel(
    t,
):
    def body(*refs):
        pass

    out_shape = jax.ShapeDtypeStruct(..., jnp.float32)
    return pl.pallas_call(body, out_shape=out_shape)(...)


# baseline (device time: 78210 ns/iter reference)
import jax
import jax.numpy as jnp
from jax import lax
from jax.experimental import pallas as pl
from jax.experimental.pallas import tpu as pltpu

N_DEV = 16
LOG_N = 4


def kernel(t):
    m, n = t.shape

    def body(x_ref, out_ref, acc_ref, send_ref, recv_ref, send_sems, recv_sems):
        my = lax.axis_index("i")

        acc_ref[...] = x_ref[...]

        for k in range(LOG_N):
            partner = my ^ (1 << k)
            send_ref[...] = acc_ref[...].astype(jnp.bfloat16)
            rdma = pltpu.make_async_remote_copy(
                src_ref=send_ref,
                dst_ref=recv_ref.at[k],
                send_sem=send_sems.at[k],
                recv_sem=recv_sems.at[k],
                device_id=(partner,),
                device_id_type=pl.DeviceIdType.MESH,
            )
            rdma.start()
            rdma.wait()
            acc_ref[...] = acc_ref[...] + recv_ref[k].astype(jnp.float32)

        s = acc_ref[...]
        r = jnp.maximum(s, 0.0)
        out_ref[...] = jnp.tanh(s) * s * s + r * r * r

    return pl.pallas_call(
        body,
        out_shape=jax.ShapeDtypeStruct((m, n), jnp.float32),
        in_specs=[pl.BlockSpec(memory_space=pltpu.VMEM)],
        out_specs=pl.BlockSpec(memory_space=pltpu.VMEM),
        scratch_shapes=[
            pltpu.VMEM((m, n), jnp.float32),
            pltpu.VMEM((m, n), jnp.bfloat16),
            pltpu.VMEM((LOG_N, m, n), jnp.bfloat16),
            pltpu.SemaphoreType.DMA((LOG_N,)),
            pltpu.SemaphoreType.DMA((LOG_N,)),
        ],
    )(t)


# device time: 47451 ns/iter; 1.6482x vs baseline; 1.6482x over previous
import jax
import jax.numpy as jnp
from jax import lax
from jax.experimental import pallas as pl
from jax.experimental.pallas import tpu as pltpu

N_DEV = 16
LOG_N = 4


def kernel(t):
    m, n = t.shape
    blk = m // N_DEV

    def body(
        x_ref,
        out_ref,
        acc_ref,
        send_ref,
        r0,
        r1,
        r2,
        r3,
        rs_send_sems,
        rs_recv_sems,
        ag_send_sems,
        ag_recv_sems,
    ):
        my = lax.axis_index("i")
        acc_ref[...] = x_ref[...]
        rs_recv = [r0, r1, r2, r3]

        start = my * 0
        for k in range(LOG_N):
            half = m >> (k + 1)
            b = (my >> k) & 1
            keep_start = pl.multiple_of(start + b * half, blk)
            send_start = pl.multiple_of(start + (1 - b) * half, blk)
            send_ref[pl.ds(0, half), :] = acc_ref[
                pl.ds(send_start, half), :
            ].astype(jnp.bfloat16)
            rdma = pltpu.make_async_remote_copy(
                src_ref=send_ref.at[pl.ds(0, half)],
                dst_ref=rs_recv[k],
                send_sem=rs_send_sems.at[k],
                recv_sem=rs_recv_sems.at[k],
                device_id=(my ^ (1 << k),),
                device_id_type=pl.DeviceIdType.MESH,
            )
            rdma.start()
            rdma.wait()
            acc_ref[pl.ds(keep_start, half), :] = acc_ref[
                pl.ds(keep_start, half), :
            ] + rs_recv[k][...].astype(jnp.float32)
            start = keep_start

        s = acc_ref[pl.ds(start, blk), :]
        r = jnp.maximum(s, 0.0)
        out_ref[pl.ds(start, blk), :] = (
            jnp.tanh(s) * s * s + r * r * r
        ).astype(jnp.bfloat16)

        for j in range(LOG_N - 1, -1, -1):
            size = m >> (j + 1)
            start = pl.multiple_of(start, blk)
            rdma = pltpu.make_async_remote_copy(
                src_ref=out_ref.at[pl.ds(start, size)],
                dst_ref=out_ref.at[pl.ds(start, size)],
                send_sem=ag_send_sems.at[j],
                recv_sem=ag_recv_sems.at[j],
                device_id=(my ^ (1 << j),),
                device_id_type=pl.DeviceIdType.MESH,
            )
            rdma.start()
            rdma.wait()
            start = start & ~size

    return pl.pallas_call(
        body,
        out_shape=jax.ShapeDtypeStruct((m, n), jnp.bfloat16),
        in_specs=[pl.BlockSpec(memory_space=pltpu.VMEM)],
        out_specs=pl.BlockSpec(memory_space=pltpu.VMEM),
        scratch_shapes=[
            pltpu.VMEM((m, n), jnp.float32),
            pltpu.VMEM((m // 2, n), jnp.bfloat16),
            pltpu.VMEM((m // 2, n), jnp.bfloat16),
            pltpu.VMEM((m // 4, n), jnp.bfloat16),
            pltpu.VMEM((m // 8, n), jnp.bfloat16),
            pltpu.VMEM((m // 16, n), jnp.bfloat16),
            pltpu.SemaphoreType.DMA((LOG_N,)),
            pltpu.SemaphoreType.DMA((LOG_N,)),
            pltpu.SemaphoreType.DMA((LOG_N,)),
            pltpu.SemaphoreType.DMA((LOG_N,)),
        ],
    )(t)


# device time: 47305 ns/iter; 1.6533x vs baseline; 1.0031x over previous
import jax
import jax.numpy as jnp
from jax import lax
from jax.experimental import pallas as pl
from jax.experimental.pallas import tpu as pltpu

N_DEV = 16
LOG_N = 4


def kernel(t):
    m, n = t.shape
    blk = m // N_DEV

    def body(
        x_ref,
        out_ref,
        acc_ref,
        send_ref,
        r0,
        r1,
        r2,
        r3,
        rs_send_sems,
        rs_recv_sems,
        ag_send_sems,
        ag_recv_sems,
    ):
        my = lax.axis_index("i")
        rs_recv = [r0, r1, r2, r3]

        start = my * 0
        for k in range(LOG_N):
            half = m >> (k + 1)
            b = (my >> k) & 1
            keep_start = pl.multiple_of(start + b * half, blk)
            send_start = pl.multiple_of(start + (1 - b) * half, blk)
            src = x_ref if k == 0 else acc_ref
            send_ref[pl.ds(0, half), :] = src[
                pl.ds(send_start, half), :
            ].astype(jnp.bfloat16)
            rdma = pltpu.make_async_remote_copy(
                src_ref=send_ref.at[pl.ds(0, half)],
                dst_ref=rs_recv[k],
                send_sem=rs_send_sems.at[k],
                recv_sem=rs_recv_sems.at[k],
                device_id=(my ^ (1 << k),),
                device_id_type=pl.DeviceIdType.MESH,
            )
            rdma.start()
            rdma.wait()
            acc_ref[pl.ds(keep_start, half), :] = src[
                pl.ds(keep_start, half), :
            ] + rs_recv[k][...].astype(jnp.float32)
            start = keep_start

        s = acc_ref[pl.ds(start, blk), :]
        r = jnp.maximum(s, 0.0)
        out_ref[pl.ds(start, blk), :] = (
            jnp.tanh(s) * s * s + r * r * r
        ).astype(jnp.bfloat16)

        for j in range(LOG_N - 1, -1, -1):
            size = m >> (j + 1)
            start = pl.multiple_of(start, blk)
            rdma = pltpu.make_async_remote_copy(
                src_ref=out_ref.at[pl.ds(start, size)],
                dst_ref=out_ref.at[pl.ds(start, size)],
                send_sem=ag_send_sems.at[j],
                recv_sem=ag_recv_sems.at[j],
                device_id=(my ^ (1 << j),),
                device_id_type=pl.DeviceIdType.MESH,
            )
            rdma.start()
            rdma.wait()
            start = start & ~size

    return pl.pallas_call(
        body,
        out_shape=jax.ShapeDtypeStruct((m, n), jnp.bfloat16),
        in_specs=[pl.BlockSpec(memory_space=pltpu.VMEM)],
        out_specs=pl.BlockSpec(memory_space=pltpu.VMEM),
        scratch_shapes=[
            pltpu.VMEM((m, n), jnp.float32),
            pltpu.VMEM((m // 2, n), jnp.bfloat16),
            pltpu.VMEM((m // 2, n), jnp.bfloat16),
            pltpu.VMEM((m // 4, n), jnp.bfloat16),
            pltpu.VMEM((m // 8, n), jnp.bfloat16),
            pltpu.VMEM((m // 16, n), jnp.bfloat16),
            pltpu.SemaphoreType.DMA((LOG_N,)),
            pltpu.SemaphoreType.DMA((LOG_N,)),
            pltpu.SemaphoreType.DMA((LOG_N,)),
            pltpu.SemaphoreType.DMA((LOG_N,)),
        ],
    )(t)


# device time: 40704 ns/iter; 1.9214x vs baseline; 1.1622x over previous
import jax
import jax.numpy as jnp
from jax import lax
from jax.experimental import pallas as pl
from jax.experimental.pallas import tpu as pltpu

N_DEV = 16
LOG_N = 4
BITS_A = (0, 1, 2, 3)
BITS_B = (2, 3, 0, 1)


def kernel(t):
    m, n = t.shape
    nc = n // 2
    blk = m // N_DEV

    def body(
        x_ref,
        out_ref,
        acc_a,
        acc_b,
        send_a,
        send_b,
        ra0,
        ra1,
        ra2,
        ra3,
        rb0,
        rb1,
        rb2,
        rb3,
        rs_ss_a,
        rs_rs_a,
        rs_ss_b,
        rs_rs_b,
        ag_ss_a,
        ag_rs_a,
        ag_ss_b,
        ag_rs_b,
    ):
        my = lax.axis_index("i")

        flows = [
            {
                "bits": BITS_A,
                "c0": 0,
                "acc": acc_a,
                "send": send_a,
                "recv": [ra0, ra1, ra2, ra3],
                "rs_ss": rs_ss_a,
                "rs_rs": rs_rs_a,
                "ag_ss": ag_ss_a,
                "ag_rs": ag_rs_a,
                "start": my * 0,
            },
            {
                "bits": BITS_B,
                "c0": nc,
                "acc": acc_b,
                "send": send_b,
                "recv": [rb0, rb1, rb2, rb3],
                "rs_ss": rs_ss_b,
                "rs_rs": rs_rs_b,
                "ag_ss": ag_ss_b,
                "ag_rs": ag_rs_b,
                "start": my * 0,
            },
        ]

        def rs_stage_and_start(fl, step):
            half = m >> (step + 1)
            k = fl["bits"][step]
            b = (my >> k) & 1
            keep = pl.multiple_of(fl["start"] + b * half, blk)
            snd = pl.multiple_of(fl["start"] + (1 - b) * half, blk)
            c0 = fl["c0"]
            if step == 0:
                fl["send"][pl.ds(0, half), :] = x_ref[
                    pl.ds(snd, half), c0 : c0 + nc
                ].astype(jnp.bfloat16)
            else:
                fl["send"][pl.ds(0, half), :] = fl["acc"][
                    pl.ds(snd, half), :
                ].astype(jnp.bfloat16)
            rdma = pltpu.make_async_remote_copy(
                src_ref=fl["send"].at[pl.ds(0, half)],
                dst_ref=fl["recv"][step],
                send_sem=fl["rs_ss"].at[step],
                recv_sem=fl["rs_rs"].at[step],
                device_id=(my ^ (1 << k),),
                device_id_type=pl.DeviceIdType.MESH,
            )
            rdma.start()
            fl["rdma"] = rdma
            fl["keep"] = keep
            fl["step"] = step

        def rs_finish(fl):
            step = fl["step"]
            half = m >> (step + 1)
            keep = fl["keep"]
            c0 = fl["c0"]
            fl["rdma"].wait()
            if step == 0:
                base = x_ref[pl.ds(keep, half), c0 : c0 + nc]
            else:
                base = fl["acc"][pl.ds(keep, half), :]
            fl["acc"][pl.ds(keep, half), :] = base + fl["recv"][step][
                ...
            ].astype(jnp.float32)
            fl["start"] = keep

        rs_stage_and_start(flows[0], 0)
        rs_stage_and_start(flows[1], 0)
        for step in range(LOG_N):
            for fl in flows:
                rs_finish(fl)
                if step + 1 < LOG_N:
                    rs_stage_and_start(fl, step + 1)

        for fl in flows:
            st = pl.multiple_of(fl["start"], blk)
            s = fl["acc"][pl.ds(st, blk), :]
            r = jnp.maximum(s, 0.0)
            out_ref[pl.ds(st, blk), pl.ds(fl["c0"], nc)] = (
                jnp.tanh(s) * s * s + r * r * r
            ).astype(jnp.bfloat16)

        def ag_start(fl, step):
            size = blk << step
            k = fl["bits"][LOG_N - 1 - step]
            st = pl.multiple_of(fl["start"], blk)
            c0 = fl["c0"]
            rdma = pltpu.make_async_remote_copy(
                src_ref=out_ref.at[pl.ds(st, size), pl.ds(c0, nc)],
                dst_ref=out_ref.at[pl.ds(st, size), pl.ds(c0, nc)],
                send_sem=fl["ag_ss"].at[step],
                recv_sem=fl["ag_rs"].at[step],
                device_id=(my ^ (1 << k),),
                device_id_type=pl.DeviceIdType.MESH,
            )
            rdma.start()
            fl["rdma"] = rdma
            fl["size"] = size

        def ag_finish(fl):
            fl["rdma"].wait()
            fl["start"] = fl["start"] & ~fl["size"]

        ag_start(flows[0], 0)
        ag_start(flows[1], 0)
        for step in range(LOG_N):
            for fl in flows:
                ag_finish(fl)
                if step + 1 < LOG_N:
                    ag_start(fl, step + 1)

    return pl.pallas_call(
        body,
        out_shape=jax.ShapeDtypeStruct((m, n), jnp.bfloat16),
        in_specs=[pl.BlockSpec(memory_space=pltpu.VMEM)],
        out_specs=pl.BlockSpec(memory_space=pltpu.VMEM),
        scratch_shapes=[
            pltpu.VMEM((m, nc), jnp.float32),
            pltpu.VMEM((m, nc), jnp.float32),
            pltpu.VMEM((m // 2, nc), jnp.bfloat16),
            pltpu.VMEM((m // 2, nc), jnp.bfloat16),
            pltpu.VMEM((m // 2, nc), jnp.bfloat16),
            pltpu.VMEM((m // 4, nc), jnp.bfloat16),
            pltpu.VMEM((m // 8, nc), jnp.bfloat16),
            pltpu.VMEM((m // 16, nc), jnp.bfloat16),
            pltpu.VMEM((m // 2, nc), jnp.bfloat16),
            pltpu.VMEM((m // 4, nc), jnp.bfloat16),
            pltpu.VMEM((m // 8, nc), jnp.bfloat16),
            pltpu.VMEM((m // 16, nc), jnp.bfloat16),
            pltpu.SemaphoreType.DMA((LOG_N,)),
            pltpu.SemaphoreType.DMA((LOG_N,)),
            pltpu.SemaphoreType.DMA((LOG_N,)),
            pltpu.SemaphoreType.DMA((LOG_N,)),
            pltpu.SemaphoreType.DMA((LOG_N,)),
            pltpu.SemaphoreType.DMA((LOG_N,)),
            pltpu.SemaphoreType.DMA((LOG_N,)),
            pltpu.SemaphoreType.DMA((LOG_N,)),
        ],
    )(t)


# device time: 34151 ns/iter; 2.2901x vs baseline; 1.1919x over previous
import jax
import jax.numpy as jnp
from jax import lax
from jax.experimental import pallas as pl
from jax.experimental.pallas import tpu as pltpu

N_DEV = 16


def kernel(t):
    m, n = t.shape
    blk = m // N_DEV

    def body(x_ref, out_ref, xbf_ref, acc_ref, rs_buf, rs_ss, rs_rs, ag_ss, ag_rs):
        my = lax.axis_index("i")
        my_row = pl.multiple_of(my * blk, blk)

        xbf_ref[...] = x_ref[...].astype(jnp.bfloat16)

        rs_sends = []
        for off in range(1, N_DEV):
            d = (my + off) % N_DEV
            rdma = pltpu.make_async_remote_copy(
                src_ref=xbf_ref.at[pl.ds(pl.multiple_of(d * blk, blk), blk)],
                dst_ref=rs_buf.at[my],
                send_sem=rs_ss.at[d],
                recv_sem=rs_rs.at[my],
                device_id=(d,),
                device_id_type=pl.DeviceIdType.MESH,
            )
            rdma.start()
            rs_sends.append(rdma)

        acc_ref[...] = x_ref[pl.ds(my_row, blk), :]
        for off in range(1, N_DEV):
            src = (my - off) % N_DEV
            recv = pltpu.make_async_remote_copy(
                src_ref=rs_buf.at[src],
                dst_ref=rs_buf.at[src],
                send_sem=rs_ss.at[src],
                recv_sem=rs_rs.at[src],
                device_id=(src,),
                device_id_type=pl.DeviceIdType.MESH,
            )
            recv.wait_recv()
            acc_ref[...] = acc_ref[...] + rs_buf[src].astype(jnp.float32)

        s = acc_ref[...]
        r = jnp.maximum(s, 0.0)
        out_ref[pl.ds(my_row, blk), :] = (
            jnp.tanh(s) * s * s + r * r * r
        ).astype(jnp.bfloat16)

        ag_sends = []
        for off in range(1, N_DEV):
            d = (my + off) % N_DEV
            rdma = pltpu.make_async_remote_copy(
                src_ref=out_ref.at[pl.ds(my_row, blk)],
                dst_ref=out_ref.at[pl.ds(my_row, blk)],
                send_sem=ag_ss.at[d],
                recv_sem=ag_rs.at[my],
                device_id=(d,),
                device_id_type=pl.DeviceIdType.MESH,
            )
            rdma.start()
            ag_sends.append(rdma)

        for off in range(1, N_DEV):
            src = (my - off) % N_DEV
            src_row = pl.multiple_of(src * blk, blk)
            recv = pltpu.make_async_remote_copy(
                src_ref=out_ref.at[pl.ds(src_row, blk)],
                dst_ref=out_ref.at[pl.ds(src_row, blk)],
                send_sem=ag_ss.at[src],
                recv_sem=ag_rs.at[src],
                device_id=(src,),
                device_id_type=pl.DeviceIdType.MESH,
            )
            recv.wait_recv()

        for rdma in rs_sends + ag_sends:
            rdma.wait_send()

    return pl.pallas_call(
        body,
        out_shape=jax.ShapeDtypeStruct((m, n), jnp.bfloat16),
        in_specs=[pl.BlockSpec(memory_space=pltpu.VMEM)],
        out_specs=pl.BlockSpec(memory_space=pltpu.VMEM),
        scratch_shapes=[
            pltpu.VMEM((m, n), jnp.bfloat16),
            pltpu.VMEM((blk, n), jnp.float32),
            pltpu.VMEM((N_DEV, blk, n), jnp.bfloat16),
            pltpu.SemaphoreType.DMA((N_DEV,)),
            pltpu.SemaphoreType.DMA((N_DEV,)),
            pltpu.SemaphoreType.DMA((N_DEV,)),
            pltpu.SemaphoreType.DMA((N_DEV,)),
        ],
    )(t)


# device time: 28413 ns/iter; 2.7526x vs baseline; 1.2019x over previous
import jax
import jax.numpy as jnp
from jax import lax
from jax.experimental import pallas as pl
from jax.experimental.pallas import tpu as pltpu

N_DEV = 16
_OFFS_SEND = list(range(1, N_DEV))
_OFFS_WAIT = list(range(1, N_DEV))


def kernel(t):
    m, n = t.shape
    blk = m // N_DEV

    def body(x_ref, out_ref, xbf_ref, acc_ref, rs_buf, rs_ss, rs_rs, ag_ss, ag_rs):
        my = lax.axis_index("i")
        my_row = pl.multiple_of(my * blk, blk)

        barrier_sem = pltpu.get_barrier_semaphore()
        pl.semaphore_signal(
            barrier_sem,
            inc=1,
            device_id=((my + 1) % N_DEV,),
            device_id_type=pl.DeviceIdType.MESH,
        )
        pl.semaphore_wait(barrier_sem, 1)

        rs_sends = []
        for off in _OFFS_SEND:
            d = (my + off) % N_DEV
            row = pl.multiple_of(d * blk, blk)
            xbf_ref[pl.ds(row, blk), :] = x_ref[pl.ds(row, blk), :].astype(
                jnp.bfloat16
            )
            rdma = pltpu.make_async_remote_copy(
                src_ref=xbf_ref.at[pl.ds(row, blk)],
                dst_ref=rs_buf.at[my],
                send_sem=rs_ss.at[d],
                recv_sem=rs_rs.at[my],
                device_id=(d,),
                device_id_type=pl.DeviceIdType.MESH,
            )
            rdma.start()
            rs_sends.append(rdma)

        acc_ref[...] = x_ref[pl.ds(my_row, blk), :]
        for off in _OFFS_WAIT:
            src = (my - off) % N_DEV
            recv = pltpu.make_async_remote_copy(
                src_ref=rs_buf.at[src],
                dst_ref=rs_buf.at[src],
                send_sem=rs_ss.at[src],
                recv_sem=rs_rs.at[src],
                device_id=(src,),
                device_id_type=pl.DeviceIdType.MESH,
            )
            recv.wait_recv()
            acc_ref[...] = acc_ref[...] + rs_buf[src].astype(jnp.float32)

        s = acc_ref[...]
        r = jnp.maximum(s, 0.0)
        out_ref[pl.ds(my_row, blk), :] = (
            jnp.tanh(s) * s * s + r * r * r
        ).astype(jnp.bfloat16)

        ag_sends = []
        for off in _OFFS_SEND:
            d = (my + off) % N_DEV
            rdma = pltpu.make_async_remote_copy(
                src_ref=out_ref.at[pl.ds(my_row, blk)],
                dst_ref=out_ref.at[pl.ds(my_row, blk)],
                send_sem=ag_ss.at[d],
                recv_sem=ag_rs.at[my],
                device_id=(d,),
                device_id_type=pl.DeviceIdType.MESH,
            )
            rdma.start()
            ag_sends.append(rdma)

        for off in _OFFS_WAIT:
            src = (my - off) % N_DEV
            src_row = pl.multiple_of(src * blk, blk)
            recv = pltpu.make_async_remote_copy(
                src_ref=out_ref.at[pl.ds(src_row, blk)],
                dst_ref=out_ref.at[pl.ds(src_row, blk)],
                send_sem=ag_ss.at[src],
                recv_sem=ag_rs.at[src],
                device_id=(src,),
                device_id_type=pl.DeviceIdType.MESH,
            )
            recv.wait_recv()

        for rdma in rs_sends + ag_sends:
            rdma.wait_send()

    return pl.pallas_call(
        body,
        out_shape=jax.ShapeDtypeStruct((m, n), jnp.bfloat16),
        in_specs=[pl.BlockSpec(memory_space=pltpu.VMEM)],
        out_specs=pl.BlockSpec(memory_space=pltpu.VMEM),
        scratch_shapes=[
            pltpu.VMEM((m, n), jnp.bfloat16),
            pltpu.VMEM((blk, n), jnp.float32),
            pltpu.VMEM((N_DEV, blk, n), jnp.bfloat16),
            pltpu.SemaphoreType.DMA((N_DEV,)),
            pltpu.SemaphoreType.DMA((N_DEV,)),
            pltpu.SemaphoreType.DMA((N_DEV,)),
            pltpu.SemaphoreType.DMA((N_DEV,)),
        ],
        compiler_params=pltpu.CompilerParams(collective_id=0),
    )(t)
